# baseline (speedup 1.0000x reference)
import functools

import jax
import jax.numpy as jnp
from jax import lax
from jax.experimental import pallas as pl
from jax.experimental.pallas import tpu as pltpu
from jax.experimental.pallas import tpu_sc as plsc

N = 10000
E = 320000
D_IN = 128
D_HID = 32
N_CLASSES = 8

NC = 2
NS = 16
LANES = 128

N_PAD = 10240
ROWS = 2560
E_PAD = ROWS * LANES
FILL = N

MR = 1280
MROWS = E_PAD // MR
MR_PER_SUB_AGG = MROWS // (NC * NS)
MR_PER_SUB_DEG = MROWS // NS
DRAIN = N_PAD // NS

_mesh = plsc.VectorSubcoreMesh(core_axis_name="c", subcore_axis_name="s")
_sc_params = pltpu.CompilerParams(use_tc_tiling_on_sc=False)
NBUF = 8


def _deg_call(e_pad, ones16, zeros16):
  nb = 4

  @functools.partial(
      pl.kernel,
      out_type=jax.ShapeDtypeStruct((NC, N_PAD, 16), jnp.float32),
      mesh=_mesh,
      scratch_types=[
          pltpu.VMEM((MR_PER_SUB_DEG, MR), jnp.int32),
          pltpu.VMEM((MR, 16), jnp.float32),
          pltpu.VMEM_SHARED((N_PAD, 16), jnp.float32),
          pltpu.SemaphoreType.DMA((4,)),
      ],
      compiler_params=_sc_params,
  )
  def k(e_hbm, ones_hbm, zeros_hbm, out_hbm, idx_v, ones_v, acc, sems):
    c = lax.axis_index("c")
    s = lax.axis_index("s")
    pltpu.sync_copy(zeros_hbm.at[pl.ds(s * DRAIN, DRAIN)],
                    acc.at[pl.ds(s * DRAIN, DRAIN)])
    pltpu.sync_copy(ones_hbm, ones_v)
    pltpu.sync_copy(e_hbm.at[c, pl.ds(s * MR_PER_SUB_DEG, MR_PER_SUB_DEG)],
                    idx_v)
    plsc.subcore_barrier()

    def add_start(j, b):
      pltpu.async_copy(ones_v, acc.at[idx_v.at[j]], sems.at[b],
                       add=True)

    def add_wait(j, b):
      pltpu.make_async_copy(ones_v, acc.at[idx_v.at[j]],
                            sems.at[b]).wait()

    for b in range(nb):
      add_start(b, b)

    @pl.loop(nb, MR_PER_SUB_DEG, step=nb)
    def _(j0):
      for b in range(nb):
        add_wait(j0 + b - nb, b)
        add_start(j0 + b, b)

    for b in range(nb):
      add_wait(MR_PER_SUB_DEG - nb + b, b)

    plsc.subcore_barrier()
    pltpu.sync_copy(acc.at[pl.ds(s * DRAIN, DRAIN)],
                    out_hbm.at[c, pl.ds(s * DRAIN, DRAIN)])

  return k(e_pad, ones16, zeros16)


def _agg_call(h_pad, e_pad, zeros, d):
  nb = 2

  @functools.partial(
      pl.kernel,
      out_type=jax.ShapeDtypeStruct((NC, N_PAD, d), jnp.float32),
      mesh=_mesh,
      scratch_types=[
          pltpu.VMEM((MR_PER_SUB_AGG, MR), jnp.int32),
          pltpu.VMEM((MR_PER_SUB_AGG, MR), jnp.int32),
          [pltpu.VMEM((MR, d), jnp.float32) for _ in range(nb)],
          pltpu.VMEM_SHARED((N_PAD, d), jnp.float32),
          pltpu.SemaphoreType.DMA((nb,)),
          pltpu.SemaphoreType.DMA((nb,)),
      ],
      compiler_params=_sc_params,
  )
  def k(h_hbm, e_hbm, zeros_hbm, out_hbm, src_v, dst_v, rows_v, acc, gs, ss):
    c = lax.axis_index("c")
    s = lax.axis_index("s")
    pltpu.sync_copy(zeros_hbm.at[pl.ds(s * DRAIN, DRAIN)],
                    acc.at[pl.ds(s * DRAIN, DRAIN)])
    base = c * (NS * MR_PER_SUB_AGG) + s * MR_PER_SUB_AGG
    pltpu.sync_copy(e_hbm.at[0, pl.ds(base, MR_PER_SUB_AGG)], src_v)
    pltpu.sync_copy(e_hbm.at[1, pl.ds(base, MR_PER_SUB_AGG)], dst_v)
    plsc.subcore_barrier()

    last = MR_PER_SUB_AGG - 1

    def g_start(j, b):
      pltpu.async_copy(h_hbm.at[src_v.at[j]], rows_v[b], gs.at[b])

    def g_wait(j, b):
      pltpu.make_async_copy(h_hbm.at[src_v.at[j]], rows_v[b],
                            gs.at[b]).wait()

    def s_start(j, b):
      pltpu.async_copy(rows_v[b], acc.at[dst_v.at[j]], ss.at[b],
                       add=True)

    def s_wait(j, b):
      pltpu.make_async_copy(rows_v[b], acc.at[dst_v.at[j]],
                            ss.at[b]).wait()

    for b in range(nb):
      g_start(b, b)

    @pl.loop(0, MR_PER_SUB_AGG, step=nb)
    def _(j0):
      for b in range(nb):
        g_wait(j0 + b, b)
        s_start(j0 + b, b)
      for b in range(nb):
        s_wait(j0 + b, b)
        g_start(jnp.minimum(j0 + nb + b, last), b)

    for b in range(nb):
      g_wait(last, b)

    plsc.subcore_barrier()
    pltpu.sync_copy(acc.at[pl.ds(s * DRAIN, DRAIN)],
                    out_hbm.at[c, pl.ds(s * DRAIN, DRAIN)])

  return k(h_pad, e_pad, zeros)


_BLK = 2048
_GRID = N_PAD // _BLK


def _inv_sqrt(deg_col):
  return jnp.where(deg_col > 0.0,
                   lax.rsqrt(jnp.maximum(deg_col, 1.0)), 0.0)


def _deg_cols(deg):
  return deg[0, :, 0:1], deg[1, :, 0:1]


_DEG_SPEC = lambda: pl.BlockSpec((NC, _BLK, 16), lambda i: (0, i, 0))


def _mm1s_body(x_ref, deg_ref, w_ref, o_ref):
  dsrc, _ = _deg_cols(deg_ref[...])
  o_ref[...] = jnp.dot(x_ref[...] * _inv_sqrt(dsrc), w_ref[...],
                       preferred_element_type=jnp.float32)


def _mm1s(feat_pad, deg, w1):
  return pl.pallas_call(
      _mm1s_body,
      grid=(_GRID,),
      in_specs=[
          pl.BlockSpec((_BLK, D_IN), lambda i: (i, 0)),
          _DEG_SPEC(),
          pl.BlockSpec((D_IN, D_HID), lambda i: (0, 0)),
      ],
      out_specs=pl.BlockSpec((_BLK, D_HID), lambda i: (i, 0)),
      out_shape=jax.ShapeDtypeStruct((N_PAD, D_HID), jnp.float32),
  )(feat_pad, deg, w1)


def _layer2_body(p_ref, deg_ref, w_ref, b_ref, o_ref):
  p = p_ref[...]
  agg = p[0] + p[1]
  dsrc, ddst = _deg_cols(deg_ref[...])
  x1 = jnp.maximum(agg * _inv_sqrt(ddst) + b_ref[...], 0.0)
  o_ref[...] = jnp.dot(x1, w_ref[...],
                       preferred_element_type=jnp.float32) * _inv_sqrt(dsrc)


def _layer2(p1, deg, w2p, b1r):
  return pl.pallas_call(
      _layer2_body,
      grid=(_GRID,),
      in_specs=[
          pl.BlockSpec((NC, _BLK, D_HID), lambda i: (0, i, 0)),
          _DEG_SPEC(),
          pl.BlockSpec((D_HID, 16), lambda i: (0, 0)),
          pl.BlockSpec((1, D_HID), lambda i: (0, 0)),
      ],
      out_specs=pl.BlockSpec((_BLK, 16), lambda i: (i, 0)),
      out_shape=jax.ShapeDtypeStruct((N_PAD, 16), jnp.float32),
  )(p1, deg, w2p, b1r)


def _final_body(p_ref, deg_ref, b_ref, o_ref):
  p = p_ref[...]
  agg = p[0] + p[1]
  _, ddst = _deg_cols(deg_ref[...])
  o_ref[...] = agg[:, :N_CLASSES] * _inv_sqrt(ddst) + b_ref[...]


def _final(p2, deg, b2r):
  return pl.pallas_call(
      _final_body,
      grid=(_GRID,),
      in_specs=[
          pl.BlockSpec((NC, _BLK, 16), lambda i: (0, i, 0)),
          _DEG_SPEC(),
          pl.BlockSpec((1, N_CLASSES), lambda i: (0, 0)),
      ],
      out_specs=pl.BlockSpec((_BLK, N_CLASSES), lambda i: (i, 0)),
      out_shape=jax.ShapeDtypeStruct((N, N_CLASSES), jnp.float32),
  )(p2, deg, b2r)


def kernel(features, edge_index, W1, b1, W2, b2):
  fill = FILL + (jnp.arange(E_PAD - E, dtype=jnp.int32) % (N_PAD - N))
  e_pad = jnp.concatenate(
      [edge_index.astype(jnp.int32),
       jnp.broadcast_to(fill, (2, E_PAD - E))], axis=1).reshape(2, MROWS, MR)
  feat_pad = jnp.pad(features, ((0, N_PAD - N), (0, 0)))
  zeros16 = jnp.zeros((N_PAD, 16), jnp.float32)
  zeros32 = jnp.zeros((N_PAD, D_HID), jnp.float32)
  ones16 = jnp.ones((MR, 16), jnp.float32)
  w2p = jnp.pad(W2, ((0, 0), (0, 16 - N_CLASSES)))
  b1r = b1.reshape(1, D_HID)
  b2r = b2.reshape(1, N_CLASSES)

  deg = _deg_call(e_pad, ones16, zeros16)
  h1n = _mm1s(feat_pad, deg, W1)
  p1 = _agg_call(h1n, e_pad, zeros32, D_HID)
  h2n = _layer2(p1, deg, w2p, b1r)
  p2 = _agg_call(h2n, e_pad, zeros16, 16)
  return _final(p2, deg, b2r)


# speedup vs baseline: 1.0718x; 1.0718x over previous
import functools

import jax
import jax.numpy as jnp
from jax import lax
from jax.experimental import pallas as pl
from jax.experimental.pallas import tpu as pltpu
from jax.experimental.pallas import tpu_sc as plsc

N = 10000
E = 320000
D_IN = 128
D_HID = 32
N_CLASSES = 8

NC = 2
NS = 16
LANES = 128

N_PAD = 10240
ROWS = 2560
E_PAD = ROWS * LANES
FILL = N

ROWS_PER_SUB_AGG = ROWS // (NC * NS)
ROWS_PER_SUB_DEG = ROWS // NS
DRAIN = N_PAD // NS

_mesh = plsc.VectorSubcoreMesh(core_axis_name="c", subcore_axis_name="s")
_sc_params = pltpu.CompilerParams(use_tc_tiling_on_sc=False)
NBUF = 8


def _deg_call(e_pad, ones16, zeros16):
  @functools.partial(
      pl.kernel,
      out_type=jax.ShapeDtypeStruct((NC, N_PAD, 16), jnp.float32),
      mesh=_mesh,
      scratch_types=[
          pltpu.VMEM((ROWS_PER_SUB_DEG, LANES), jnp.int32),
          pltpu.VMEM((LANES, 16), jnp.float32),
          pltpu.VMEM_SHARED((N_PAD, 16), jnp.float32),
          pltpu.SemaphoreType.DMA((NBUF,)),
      ],
  )
  def k(e_hbm, ones_hbm, zeros_hbm, out_hbm, idx_v, ones_v, acc, sems):
    c = lax.axis_index("c")
    s = lax.axis_index("s")
    pltpu.sync_copy(zeros_hbm.at[pl.ds(s * DRAIN, DRAIN)],
                    acc.at[pl.ds(s * DRAIN, DRAIN)])
    pltpu.sync_copy(ones_hbm, ones_v)
    pltpu.sync_copy(e_hbm.at[c, pl.ds(s * ROWS_PER_SUB_DEG, ROWS_PER_SUB_DEG)],
                    idx_v)
    plsc.subcore_barrier()

    def add_start(j, b):
      pltpu.async_copy(ones_v, acc.at[idx_v.at[j]], sems.at[b], add=True)

    def add_wait(j, b):
      pltpu.make_async_copy(ones_v, acc.at[idx_v.at[j]], sems.at[b]).wait()

    for b in range(NBUF):
      add_start(b, b)

    @pl.loop(NBUF, ROWS_PER_SUB_DEG, step=NBUF)
    def _(j0):
      for b in range(NBUF):
        add_wait(j0 + b - NBUF, b)
        add_start(j0 + b, b)

    for b in range(NBUF):
      add_wait(ROWS_PER_SUB_DEG - NBUF + b, b)

    plsc.subcore_barrier()
    pltpu.sync_copy(acc.at[pl.ds(s * DRAIN, DRAIN)],
                    out_hbm.at[c, pl.ds(s * DRAIN, DRAIN)])

  return k(e_pad, ones16, zeros16)


def _agg_call(h_pad, e_pad, zeros, d):
  @functools.partial(
      pl.kernel,
      out_type=jax.ShapeDtypeStruct((NC, N_PAD, d), jnp.float32),
      mesh=_mesh,
      scratch_types=[
          pltpu.VMEM((ROWS_PER_SUB_AGG, LANES), jnp.int32),
          pltpu.VMEM((ROWS_PER_SUB_AGG, LANES), jnp.int32),
          [pltpu.VMEM((LANES, d), jnp.float32) for _ in range(NBUF)],
          pltpu.VMEM_SHARED((N_PAD, d), jnp.float32),
          pltpu.SemaphoreType.DMA((NBUF,)),
          pltpu.SemaphoreType.DMA((NBUF,)),
      ],
      compiler_params=_sc_params,
  )
  def k(h_hbm, e_hbm, zeros_hbm, out_hbm, src_v, dst_v, rows_v, acc, gs, ss):
    c = lax.axis_index("c")
    s = lax.axis_index("s")
    pltpu.sync_copy(zeros_hbm.at[pl.ds(s * DRAIN, DRAIN)],
                    acc.at[pl.ds(s * DRAIN, DRAIN)])
    base = c * (NS * ROWS_PER_SUB_AGG) + s * ROWS_PER_SUB_AGG
    pltpu.sync_copy(e_hbm.at[0, pl.ds(base, ROWS_PER_SUB_AGG)], src_v)
    pltpu.sync_copy(e_hbm.at[1, pl.ds(base, ROWS_PER_SUB_AGG)], dst_v)
    plsc.subcore_barrier()

    last = ROWS_PER_SUB_AGG - 1

    def g_start(j, b):
      pltpu.async_copy(h_hbm.at[src_v.at[j]], rows_v[b], gs.at[b])

    def g_wait(j, b):
      pltpu.make_async_copy(h_hbm.at[src_v.at[j]], rows_v[b], gs.at[b]).wait()

    def s_start(j, b):
      pltpu.async_copy(rows_v[b], acc.at[dst_v.at[j]], ss.at[b], add=True)

    def s_wait(j, b):
      pltpu.make_async_copy(rows_v[b], acc.at[dst_v.at[j]], ss.at[b]).wait()

    for b in range(NBUF):
      g_start(b, b)

    @pl.loop(0, ROWS_PER_SUB_AGG, step=NBUF)
    def _(j0):
      for b in range(NBUF):
        g_wait(j0 + b, b)
        s_start(j0 + b, b)
      for b in range(NBUF):
        s_wait(j0 + b, b)
        g_start(jnp.minimum(j0 + NBUF + b, last), b)

    for b in range(NBUF):
      g_wait(last, b)

    plsc.subcore_barrier()
    pltpu.sync_copy(acc.at[pl.ds(s * DRAIN, DRAIN)],
                    out_hbm.at[c, pl.ds(s * DRAIN, DRAIN)])

  return k(h_pad, e_pad, zeros)


_BLK = 2048
_GRID = N_PAD // _BLK


def _inv_sqrt(deg_col):
  return jnp.where(deg_col > 0.0,
                   lax.rsqrt(jnp.maximum(deg_col, 1.0)), 0.0)


def _deg_cols(deg):
  return deg[0, :, 0:1], deg[1, :, 0:1]


_DEG_SPEC = lambda: pl.BlockSpec((NC, _BLK, 16), lambda i: (0, i, 0))


def _mm1s_body(x_ref, deg_ref, w_ref, o_ref):
  dsrc, _ = _deg_cols(deg_ref[...])
  o_ref[...] = jnp.dot(x_ref[...] * _inv_sqrt(dsrc), w_ref[...],
                       preferred_element_type=jnp.float32)


def _mm1s(feat_pad, deg, w1):
  return pl.pallas_call(
      _mm1s_body,
      grid=(_GRID,),
      in_specs=[
          pl.BlockSpec((_BLK, D_IN), lambda i: (i, 0)),
          _DEG_SPEC(),
          pl.BlockSpec((D_IN, D_HID), lambda i: (0, 0)),
      ],
      out_specs=pl.BlockSpec((_BLK, D_HID), lambda i: (i, 0)),
      out_shape=jax.ShapeDtypeStruct((N_PAD, D_HID), jnp.float32),
  )(feat_pad, deg, w1)


def _layer2_body(p_ref, deg_ref, w_ref, b_ref, o_ref):
  p = p_ref[...]
  agg = p[0] + p[1]
  dsrc, ddst = _deg_cols(deg_ref[...])
  x1 = jnp.maximum(agg * _inv_sqrt(ddst) + b_ref[...], 0.0)
  o_ref[...] = jnp.dot(x1, w_ref[...],
                       preferred_element_type=jnp.float32) * _inv_sqrt(dsrc)


def _layer2(p1, deg, w2p, b1r):
  return pl.pallas_call(
      _layer2_body,
      grid=(_GRID,),
      in_specs=[
          pl.BlockSpec((NC, _BLK, D_HID), lambda i: (0, i, 0)),
          _DEG_SPEC(),
          pl.BlockSpec((D_HID, 16), lambda i: (0, 0)),
          pl.BlockSpec((1, D_HID), lambda i: (0, 0)),
      ],
      out_specs=pl.BlockSpec((_BLK, 16), lambda i: (i, 0)),
      out_shape=jax.ShapeDtypeStruct((N_PAD, 16), jnp.float32),
  )(p1, deg, w2p, b1r)


def _final_body(p_ref, deg_ref, b_ref, o_ref):
  p = p_ref[...]
  agg = p[0] + p[1]
  _, ddst = _deg_cols(deg_ref[...])
  o_ref[...] = agg[:, :N_CLASSES] * _inv_sqrt(ddst) + b_ref[...]


def _final(p2, deg, b2r):
  return pl.pallas_call(
      _final_body,
      grid=(_GRID,),
      in_specs=[
          pl.BlockSpec((NC, _BLK, 16), lambda i: (0, i, 0)),
          _DEG_SPEC(),
          pl.BlockSpec((1, N_CLASSES), lambda i: (0, 0)),
      ],
      out_specs=pl.BlockSpec((_BLK, N_CLASSES), lambda i: (i, 0)),
      out_shape=jax.ShapeDtypeStruct((N, N_CLASSES), jnp.float32),
  )(p2, deg, b2r)


def kernel(features, edge_index, W1, b1, W2, b2):
  fill = FILL + (jnp.arange(E_PAD - E, dtype=jnp.int32) % (N_PAD - N))
  e_pad = jnp.concatenate(
      [edge_index.astype(jnp.int32),
       jnp.broadcast_to(fill, (2, E_PAD - E))], axis=1).reshape(2, ROWS, LANES)
  feat_pad = jnp.pad(features, ((0, N_PAD - N), (0, 0)))
  zeros16 = jnp.zeros((N_PAD, 16), jnp.float32)
  zeros32 = jnp.zeros((N_PAD, D_HID), jnp.float32)
  ones16 = jnp.ones((LANES, 16), jnp.float32)
  w2p = jnp.pad(W2, ((0, 0), (0, 16 - N_CLASSES)))
  b1r = b1.reshape(1, D_HID)
  b2r = b2.reshape(1, N_CLASSES)

  deg = _deg_call(e_pad, ones16, zeros16)
  h1n = _mm1s(feat_pad, deg, W1)
  p1 = _agg_call(h1n, e_pad, zeros32, D_HID)
  h2n = _layer2(p1, deg, w2p, b1r)
  p2 = _agg_call(h2n, e_pad, zeros16, 16)
  return _final(p2, deg, b2r)


# speedup vs baseline: 1.1019x; 1.0280x over previous
import functools

import jax
import jax.numpy as jnp
from jax import lax
from jax.experimental import pallas as pl
from jax.experimental.pallas import tpu as pltpu
from jax.experimental.pallas import tpu_sc as plsc

N = 10000
E = 320000
D_IN = 128
D_HID = 32
N_CLASSES = 8

NC = 2
NS = 16
LANES = 128

N_PAD = 10240
ROWS = 2560
E_PAD = ROWS * LANES
FILL = N

ROWS_PER_SUB_AGG = ROWS // (NC * NS)
ROWS_PER_SUB_DEG = ROWS // NS
DRAIN = N_PAD // NS

_mesh = plsc.VectorSubcoreMesh(core_axis_name="c", subcore_axis_name="s")
_sc_params = pltpu.CompilerParams(use_tc_tiling_on_sc=False)
NBUF = 8


def _deg_call(e_pad, ones16, zeros16):
  @functools.partial(
      pl.kernel,
      out_type=jax.ShapeDtypeStruct((NC, N_PAD, 16), jnp.float32),
      mesh=_mesh,
      scratch_types=[
          pltpu.VMEM((ROWS_PER_SUB_DEG, LANES), jnp.int32),
          pltpu.VMEM((LANES, 16), jnp.float32),
          pltpu.VMEM_SHARED((N_PAD, 16), jnp.float32),
          pltpu.SemaphoreType.DMA((NBUF,)),
      ],
      compiler_params=_sc_params,
  )
  def k(e_hbm, ones_hbm, zeros_hbm, out_hbm, idx_v, ones_v, acc, sems):
    c = lax.axis_index("c")
    s = lax.axis_index("s")
    pltpu.sync_copy(zeros_hbm.at[pl.ds(s * DRAIN, DRAIN)],
                    acc.at[pl.ds(s * DRAIN, DRAIN)])
    pltpu.sync_copy(ones_hbm, ones_v)
    pltpu.sync_copy(e_hbm.at[c, pl.ds(s * ROWS_PER_SUB_DEG, ROWS_PER_SUB_DEG)],
                    idx_v)
    plsc.subcore_barrier()

    def add_start(j, b):
      pltpu.async_copy(ones_v, acc.at[idx_v.at[j]], sems.at[b], add=True)

    def add_wait(j, b):
      pltpu.make_async_copy(ones_v, acc.at[idx_v.at[j]], sems.at[b]).wait()

    for b in range(NBUF):
      add_start(b, b)

    @pl.loop(NBUF, ROWS_PER_SUB_DEG, step=NBUF)
    def _(j0):
      for b in range(NBUF):
        add_wait(j0 + b - NBUF, b)
        add_start(j0 + b, b)

    for b in range(NBUF):
      add_wait(ROWS_PER_SUB_DEG - NBUF + b, b)

    plsc.subcore_barrier()
    pltpu.sync_copy(acc.at[pl.ds(s * DRAIN, DRAIN)],
                    out_hbm.at[c, pl.ds(s * DRAIN, DRAIN)])

  return k(e_pad, ones16, zeros16)


def _agg_call(h_pad, e_pad, zeros, d):
  @functools.partial(
      pl.kernel,
      out_type=jax.ShapeDtypeStruct((NC, N_PAD, d), jnp.float32),
      mesh=_mesh,
      scratch_types=[
          pltpu.VMEM((ROWS_PER_SUB_AGG, LANES), jnp.int32),
          pltpu.VMEM((ROWS_PER_SUB_AGG, LANES), jnp.int32),
          [pltpu.VMEM((LANES, d), jnp.float32) for _ in range(NBUF)],
          pltpu.VMEM_SHARED((N_PAD, d), jnp.float32),
          pltpu.SemaphoreType.DMA((NBUF,)),
          pltpu.SemaphoreType.DMA((NBUF,)),
      ],
      compiler_params=_sc_params,
  )
  def k(h_hbm, e_hbm, zeros_hbm, out_hbm, src_v, dst_v, rows_v, acc, gs, ss):
    c = lax.axis_index("c")
    s = lax.axis_index("s")
    pltpu.sync_copy(zeros_hbm.at[pl.ds(s * DRAIN, DRAIN)],
                    acc.at[pl.ds(s * DRAIN, DRAIN)])
    base = c * (NS * ROWS_PER_SUB_AGG) + s * ROWS_PER_SUB_AGG
    pltpu.sync_copy(e_hbm.at[0, pl.ds(base, ROWS_PER_SUB_AGG)], src_v)
    pltpu.sync_copy(e_hbm.at[1, pl.ds(base, ROWS_PER_SUB_AGG)], dst_v)
    plsc.subcore_barrier()

    last = ROWS_PER_SUB_AGG - 1

    def g_start(j, b):
      pltpu.async_copy(h_hbm.at[src_v.at[j]], rows_v[b], gs.at[b])

    def g_wait(j, b):
      pltpu.make_async_copy(h_hbm.at[src_v.at[j]], rows_v[b], gs.at[b]).wait()

    def s_start(j, b):
      pltpu.async_copy(rows_v[b], acc.at[dst_v.at[j]], ss.at[b], add=True)

    def s_wait(j, b):
      pltpu.make_async_copy(rows_v[b], acc.at[dst_v.at[j]], ss.at[b]).wait()

    for b in range(NBUF):
      g_start(b, b)

    @pl.loop(0, ROWS_PER_SUB_AGG, step=NBUF)
    def _(j0):
      for b in range(NBUF):
        g_wait(j0 + b, b)
        s_start(j0 + b, b)
      for b in range(NBUF):
        s_wait(j0 + b, b)
        g_start(jnp.minimum(j0 + NBUF + b, last), b)

    for b in range(NBUF):
      g_wait(last, b)

    plsc.subcore_barrier()
    pltpu.sync_copy(acc.at[pl.ds(s * DRAIN, DRAIN)],
                    out_hbm.at[c, pl.ds(s * DRAIN, DRAIN)])

  return k(h_pad, e_pad, zeros)


_BLK = 2048
_GRID = N_PAD // _BLK


def _inv_sqrt(deg_col):
  return jnp.where(deg_col > 0.0,
                   lax.rsqrt(jnp.maximum(deg_col, 1.0)), 0.0)


def _deg_cols(deg):
  return deg[0, :, 0:1], deg[1, :, 0:1]


_DEG_SPEC = lambda: pl.BlockSpec((NC, _BLK, 16), lambda i: (0, i, 0))


def _mm1s_body(x_ref, deg_ref, w_ref, o_ref):
  dsrc, _ = _deg_cols(deg_ref[...])
  o_ref[...] = jnp.dot(x_ref[...] * _inv_sqrt(dsrc), w_ref[...],
                       preferred_element_type=jnp.float32)


def _mm1s(feat_pad, deg, w1):
  return pl.pallas_call(
      _mm1s_body,
      grid=(_GRID,),
      in_specs=[
          pl.BlockSpec((_BLK, D_IN), lambda i: (i, 0)),
          _DEG_SPEC(),
          pl.BlockSpec((D_IN, D_HID), lambda i: (0, 0)),
      ],
      out_specs=pl.BlockSpec((_BLK, D_HID), lambda i: (i, 0)),
      out_shape=jax.ShapeDtypeStruct((N_PAD, D_HID), jnp.float32),
  )(feat_pad, deg, w1)


def _layer2_body(p_ref, deg_ref, w_ref, b_ref, o_ref):
  p = p_ref[...]
  agg = p[0] + p[1]
  dsrc, ddst = _deg_cols(deg_ref[...])
  x1 = jnp.maximum(agg * _inv_sqrt(ddst) + b_ref[...], 0.0)
  o_ref[...] = jnp.dot(x1, w_ref[...],
                       preferred_element_type=jnp.float32) * _inv_sqrt(dsrc)


def _layer2(p1, deg, w2p, b1r):
  return pl.pallas_call(
      _layer2_body,
      grid=(_GRID,),
      in_specs=[
          pl.BlockSpec((NC, _BLK, D_HID), lambda i: (0, i, 0)),
          _DEG_SPEC(),
          pl.BlockSpec((D_HID, 16), lambda i: (0, 0)),
          pl.BlockSpec((1, D_HID), lambda i: (0, 0)),
      ],
      out_specs=pl.BlockSpec((_BLK, 16), lambda i: (i, 0)),
      out_shape=jax.ShapeDtypeStruct((N_PAD, 16), jnp.float32),
  )(p1, deg, w2p, b1r)


def _final_body(p_ref, deg_ref, b_ref, o_ref):
  p = p_ref[...]
  agg = p[0] + p[1]
  _, ddst = _deg_cols(deg_ref[...])
  o_ref[...] = agg[:, :N_CLASSES] * _inv_sqrt(ddst) + b_ref[...]


def _final(p2, deg, b2r):
  return pl.pallas_call(
      _final_body,
      grid=(_GRID,),
      in_specs=[
          pl.BlockSpec((NC, _BLK, 16), lambda i: (0, i, 0)),
          _DEG_SPEC(),
          pl.BlockSpec((1, N_CLASSES), lambda i: (0, 0)),
      ],
      out_specs=pl.BlockSpec((_BLK, N_CLASSES), lambda i: (i, 0)),
      out_shape=jax.ShapeDtypeStruct((N, N_CLASSES), jnp.float32),
  )(p2, deg, b2r)


def kernel(features, edge_index, W1, b1, W2, b2):
  fill = FILL + (jnp.arange(E_PAD - E, dtype=jnp.int32) % (N_PAD - N))
  e_pad = jnp.concatenate(
      [edge_index.astype(jnp.int32),
       jnp.broadcast_to(fill, (2, E_PAD - E))], axis=1).reshape(2, ROWS, LANES)
  feat_pad = jnp.pad(features, ((0, N_PAD - N), (0, 0)))
  zeros16 = jnp.zeros((N_PAD, 16), jnp.float32)
  zeros32 = jnp.zeros((N_PAD, D_HID), jnp.float32)
  ones16 = jnp.ones((LANES, 16), jnp.float32)
  w2p = jnp.pad(W2, ((0, 0), (0, 16 - N_CLASSES)))
  b1r = b1.reshape(1, D_HID)
  b2r = b2.reshape(1, N_CLASSES)

  deg = _deg_call(e_pad, ones16, zeros16)
  h1n = _mm1s(feat_pad, deg, W1)
  p1 = _agg_call(h1n, e_pad, zeros32, D_HID)
  h2n = _layer2(p1, deg, w2p, b1r)
  p2 = _agg_call(h2n, e_pad, zeros16, 16)
  return _final(p2, deg, b2r)
